# baseline (speedup 1.0000x reference)
import functools
import math

import jax
import jax.numpy as jnp
from jax import lax
from jax.experimental import pallas as pl
from jax.experimental.pallas import tpu as pltpu
from jax.experimental.pallas import tpu_sc as plsc

K_TOP = 2
CAP_FACTOR = 1.0
AUX_COEF = 0.01


def _routing_body(x_ref, wg_ref, scidx_ref, gidx_ref, weff_ref, aux_ref,
                  xbf_ref, carry_ref, pacc_ref, *, cap, E, TM, N):
    i = pl.program_id(0)

    @pl.when(i == 0)
    def _init():
        carry_ref[...] = jnp.zeros_like(carry_ref)
        pacc_ref[...] = jnp.zeros_like(pacc_ref)

    x = x_ref[...]
    xbf_ref[...] = x.astype(jnp.bfloat16)
    wg = wg_ref[...]
    logits = jnp.dot(x, wg, preferred_element_type=jnp.float32)
    mx = jnp.max(logits, axis=1, keepdims=True)
    ex = jnp.exp(logits - mx)
    probs = ex / jnp.sum(ex, axis=1, keepdims=True)

    iota_e = lax.broadcasted_iota(jnp.int32, (TM, E), 1)
    m1 = jnp.max(probs, axis=1, keepdims=True)
    am1 = jnp.min(jnp.where(probs >= m1, iota_e, E), axis=1, keepdims=True)
    masked = jnp.where(iota_e == am1, -jnp.inf, probs)
    m2 = jnp.max(masked, axis=1, keepdims=True)
    am2 = jnp.min(jnp.where(masked >= m2, iota_e, E), axis=1, keepdims=True)
    s = m1 + m2
    w0 = m1 / (s + 1e-9)
    w1 = m2 / (s + 1e-9)

    oh0 = (iota_e == am1).astype(jnp.float32)
    oh1 = (iota_e == am2).astype(jnp.float32)
    c = oh0 + oh1

    r_i = lax.broadcasted_iota(jnp.int32, (TM, TM), 0)
    c_i = lax.broadcasted_iota(jnp.int32, (TM, TM), 1)
    ltri = (c_i < r_i).astype(jnp.float32)
    excl = jnp.dot(ltri, c, preferred_element_type=jnp.float32)
    carry = carry_ref[...]
    base = excl + carry
    pos0 = jnp.sum(base * oh0, axis=1, keepdims=True).astype(jnp.int32)
    pos1 = jnp.sum(base * oh1, axis=1, keepdims=True).astype(jnp.int32)
    kept0 = pos0 < cap
    kept1 = pos1 < cap

    tok = i * TM + lax.broadcasted_iota(jnp.int32, (TM, 1), 0)
    dump0 = E * cap + (2 * tok) % 16
    dump1 = E * cap + (2 * tok + 1) % 16
    sc0 = jnp.where(kept0, am1 * cap + pos0, dump0)
    sc1 = jnp.where(kept1, am2 * cap + pos1, dump1)
    zr = E * cap
    g0 = jnp.where(kept0, am1 * cap + pos0, zr)
    g1 = jnp.where(kept1, am2 * cap + pos1, zr)
    we0 = w0 * kept0.astype(jnp.float32)
    we1 = w1 * kept1.astype(jnp.float32)

    scidx_ref[...] = jnp.concatenate([sc0, sc1], axis=1)
    gidx_ref[...] = jnp.concatenate([g0, g1], axis=1)
    weff_ref[...] = jnp.concatenate([we0, we1], axis=1)

    new_carry = carry + jnp.sum(c, axis=0, keepdims=True)
    carry_ref[...] = new_carry
    pacc = pacc_ref[...] + jnp.sum(probs, axis=0, keepdims=True)
    pacc_ref[...] = pacc

    load = jnp.minimum(new_carry, float(cap))
    fa = load / (jnp.sum(load) + 1e-9)
    fe = pacc / N
    fe = fe / (jnp.sum(fe) + 1e-9)
    lb = jnp.mean((fe - fa) ** 2)
    aux_ref[...] = jnp.full((1, 1), AUX_COEF, jnp.float32) * lb


def _routing(xf, w_gate, *, cap, interpret=False):
    N, D = xf.shape
    E = w_gate.shape[1]
    TM = 256
    body = functools.partial(_routing_body, cap=cap, E=E, TM=TM, N=N)
    return pl.pallas_call(
        body,
        grid=(N // TM,),
        in_specs=[
            pl.BlockSpec((TM, D), lambda i: (i, 0)),
            pl.BlockSpec((D, E), lambda i: (0, 0)),
        ],
        out_specs=[
            pl.BlockSpec((TM, K_TOP), lambda i: (i, 0)),
            pl.BlockSpec((TM, K_TOP), lambda i: (i, 0)),
            pl.BlockSpec((TM, K_TOP), lambda i: (i, 0)),
            pl.BlockSpec((1, 1), lambda i: (0, 0)),
            pl.BlockSpec((TM, D), lambda i: (i, 0)),
        ],
        out_shape=[
            jax.ShapeDtypeStruct((N, K_TOP), jnp.int32),
            jax.ShapeDtypeStruct((N, K_TOP), jnp.int32),
            jax.ShapeDtypeStruct((N, K_TOP), jnp.float32),
            jax.ShapeDtypeStruct((1, 1), jnp.float32),
            jax.ShapeDtypeStruct((N, D), jnp.bfloat16),
        ],
        scratch_shapes=[
            pltpu.VMEM((1, E), jnp.float32),
            pltpu.VMEM((1, E), jnp.float32),
        ],
        compiler_params=pltpu.CompilerParams(
            dimension_semantics=("arbitrary",)),
        interpret=interpret,
    )(xf, w_gate)


def _dispatch(scidx, weff, xbf, *, n_rows):
    N, D = xbf.shape
    NK = scidx.shape[0]
    info = plsc.get_sparse_core_info()
    NW = info.num_cores * info.num_subcores
    NC = info.num_cores
    TBL = n_rows + 16
    rows_per_tile = n_rows // NW
    CH = 32
    NCH = rows_per_tile // CH

    mesh = plsc.VectorSubcoreMesh(core_axis_name="c", subcore_axis_name="s")

    @functools.partial(
        pl.kernel,
        mesh=mesh,
        out_type=[
            jax.ShapeDtypeStruct((n_rows, D), jnp.int32),
            jax.ShapeDtypeStruct((n_rows,), jnp.float32),
        ],
        scratch_types=[
            pltpu.VMEM((NK,), jnp.int32),
            pltpu.VMEM((NK,), jnp.float32),
            pltpu.VMEM((TBL,), jnp.int32),
            pltpu.VMEM((TBL,), jnp.float32),
            pltpu.VMEM((2, CH, D), jnp.int32),
            pltpu.SemaphoreType.DMA,
            pltpu.SemaphoreType.DMA,
            pltpu.SemaphoreType.DMA,
            pltpu.SemaphoreType.DMA,
        ],
        compiler_params=pltpu.CompilerParams(needs_layout_passes=False),
    )
    def k(scidx_hbm, weff_hbm, x_hbm, xd_hbm, wtab_hbm,
          idx_v, wf_v, tbl_v, wtb_v, rows_v, sg0, sg1, sw0, sw1):
        wid = lax.axis_index("s") * NC + lax.axis_index("c")
        pltpu.sync_copy(scidx_hbm, idx_v)
        pltpu.sync_copy(weff_hbm, wf_v)

        zi16 = jnp.zeros((16,), jnp.int32)
        zf16 = jnp.zeros((16,), jnp.float32)

        def zbody(j, _):
            tbl_v[pl.ds(j * 16, 16)] = zi16
            wtb_v[pl.ds(j * 16, 16)] = zf16
            return 0

        lax.fori_loop(0, TBL // 16, zbody, 0, unroll=4)

        def sbody(j, _):
            idx = idx_v[pl.ds(j * 16, 16)]
            vals = (lax.iota(jnp.int32, 16) + j * 16) // K_TOP
            plsc.store_scatter(tbl_v, [idx], vals)
            plsc.store_scatter(wtb_v, [idx], wf_v[pl.ds(j * 16, 16)])
            return 0

        lax.fori_loop(0, NK // 16, sbody, 0, unroll=4)

        base = wid * rows_per_tile
        pltpu.sync_copy(wtb_v.at[pl.ds(base, rows_per_tile)],
                        wtab_hbm.at[pl.ds(base, rows_per_tile)])

        sg = (sg0, sg1)
        sw = (sw0, sw1)

        def gstart(ci, b):
            return pltpu.async_copy(
                x_hbm.at[tbl_v.at[pl.ds(base + ci * CH, CH)]],
                rows_v.at[b], sg[b])

        gc = [None, None]
        wc = [None, None]
        gc[0] = gstart(0, 0)
        for ci in range(NCH):
            b = ci % 2
            nb = (ci + 1) % 2
            gc[b].wait()
            if ci + 1 < NCH:
                if wc[nb] is not None:
                    wc[nb].wait()
                gc[nb] = gstart(ci + 1, nb)
            wc[b] = pltpu.async_copy(
                rows_v.at[b], xd_hbm.at[pl.ds(base + ci * CH, CH)], sw[b])
        for b in range(2):
            if wc[b] is not None:
                wc[b].wait()

    return k(scidx, weff, xbf)


def _ffn_body(x_ref, w_ref, w1_ref, b1_ref, w2_ref, b2_ref, out_ref,
              acc_ref, *, E, NF):
    e = pl.program_id(0)
    f = pl.program_id(1)

    @pl.when(e == E)
    def _zeros():
        out_ref[0] = jnp.zeros_like(out_ref[0])

    @pl.when(e < E)
    def _compute():
        x = x_ref[0]
        h = jnp.dot(x, w1_ref[0], preferred_element_type=jnp.float32)
        h = h + b1_ref[0, 0]
        h = 0.5 * h * (1.0 + lax.erf(h * (1.0 / math.sqrt(2.0))))
        y = jnp.dot(h, w2_ref[0], preferred_element_type=jnp.float32)

        @pl.when(f == 0)
        def _first():
            acc_ref[...] = y + b2_ref[0]

        @pl.when(f > 0)
        def _rest():
            acc_ref[...] = acc_ref[...] + y

        @pl.when(f == NF - 1)
        def _emit():
            out_ref[0] = (acc_ref[...] * w_ref[0]).astype(jnp.bfloat16)


def _ffn(xd, wtab, W1, b1, W2, b2, *, interpret=False):
    E, cap, D = xd.shape
    FF = W1.shape[2]
    TF = 1024 if FF % 1024 == 0 else FF
    NF = FF // TF
    ce = lambda e: jnp.minimum(e, E - 1)
    body = functools.partial(_ffn_body, E=E, NF=NF)
    return pl.pallas_call(
        body,
        grid=(E + 1, NF),
        in_specs=[
            pl.BlockSpec((1, cap, D), lambda e, f: (ce(e), 0, 0)),
            pl.BlockSpec((1, cap, 1), lambda e, f: (ce(e), 0, 0)),
            pl.BlockSpec((1, D, TF), lambda e, f: (ce(e), 0, f)),
            pl.BlockSpec((1, 1, 1, TF), lambda e, f: (ce(e), f, 0, 0)),
            pl.BlockSpec((1, TF, D), lambda e, f: (ce(e), f, 0)),
            pl.BlockSpec((1, 1, D), lambda e, f: (ce(e), 0, 0)),
        ],
        out_specs=pl.BlockSpec((1, cap, D), lambda e, f: (e, 0, 0)),
        out_shape=jax.ShapeDtypeStruct((E + 1, cap, D), jnp.bfloat16),
        scratch_shapes=[pltpu.VMEM((cap, D), jnp.float32)],
        compiler_params=pltpu.CompilerParams(
            dimension_semantics=("arbitrary", "arbitrary")),
        interpret=interpret,
    )(xd, wtab.reshape(E, cap, 1), W1, b1.reshape(E, NF, 1, TF), W2,
      b2.reshape(E, 1, D))


def _combine(yd, gidx, *, N, D):
    info = plsc.get_sparse_core_info()
    NW = info.num_cores * info.num_subcores
    NC = info.num_cores
    TPT = N // NW
    CT = 16
    NCH = TPT // CT
    NV = D // 16

    mesh = plsc.VectorSubcoreMesh(core_axis_name="c", subcore_axis_name="s")

    @functools.partial(
        pl.kernel,
        mesh=mesh,
        out_type=jax.ShapeDtypeStruct((N, D), jnp.int32),
        scratch_types=[
            pltpu.VMEM((K_TOP * TPT,), jnp.int32),
            pltpu.VMEM((2, K_TOP * CT, D), jnp.int32),
            pltpu.VMEM((2, CT, D), jnp.int32),
            pltpu.SemaphoreType.DMA,
            pltpu.SemaphoreType.DMA,
            pltpu.SemaphoreType.DMA,
            pltpu.SemaphoreType.DMA,
        ],
        compiler_params=pltpu.CompilerParams(needs_layout_passes=False),
    )
    def k(yd_hbm, gidx_hbm, out_hbm, idx_v, rows_v, out_v, sg0, sg1, sw0, sw1):
        wid = lax.axis_index("s") * NC + lax.axis_index("c")
        tbase = wid * TPT
        pltpu.sync_copy(gidx_hbm.at[pl.ds(K_TOP * tbase, K_TOP * TPT)], idx_v)

        sg = (sg0, sg1)
        sw = (sw0, sw1)

        def gstart(ci, b):
            return pltpu.async_copy(
                yd_hbm.at[idx_v.at[pl.ds(ci * K_TOP * CT, K_TOP * CT)]],
                rows_v.at[b], sg[b])

        gc = [None, None]
        wc = [None, None]
        gc[0] = gstart(0, 0)
        for ci in range(NCH):
            b = ci % 2
            nb = (ci + 1) % 2
            gc[b].wait()
            if ci + 1 < NCH:
                gc[nb] = gstart(ci + 1, nb)
            if wc[b] is not None:
                wc[b].wait()

            def tbody(t, _, b=b):
                def vbody(v, _):
                    r0 = plsc.bitcast(
                        rows_v[b, 2 * t, pl.ds(v * 16, 16)], jnp.bfloat16)
                    r1 = plsc.bitcast(
                        rows_v[b, 2 * t + 1, pl.ds(v * 16, 16)], jnp.bfloat16)
                    out_v[b, t, pl.ds(v * 16, 16)] = plsc.bitcast(
                        r0 + r1, jnp.int32)
                    return 0

                lax.fori_loop(0, NV, vbody, 0, unroll=8)
                return 0

            lax.fori_loop(0, CT, tbody, 0)
            wc[b] = pltpu.async_copy(
                out_v.at[b], out_hbm.at[pl.ds(tbase + ci * CT, CT)], sw[b])
        for b in range(2):
            if wc[b] is not None:
                wc[b].wait()

    return k(yd, gidx)


def kernel(x, w_gate, W1, b1, W2, b2):
    B, T, D = x.shape
    N = B * T
    E = w_gate.shape[1]
    cap = max(1, int(CAP_FACTOR * N * max(1, K_TOP) / E + 0.9999))
    xf = x.reshape(N, D)

    scidx, gidx, weff, aux, xbf = _routing(xf, w_gate, cap=cap)
    DW = D // 2
    xbf32 = lax.bitcast_convert_type(
        xbf.reshape(N, DW, 2), jnp.int32)
    xd32, wtab = _dispatch(scidx.reshape(N * K_TOP), weff.reshape(N * K_TOP),
                           xbf32, n_rows=E * cap)
    xd = lax.bitcast_convert_type(xd32, jnp.bfloat16).reshape(E, cap, D)
    yd = _ffn(xd, wtab.reshape(E, cap), W1, b1, W2, b2)
    yd32 = lax.bitcast_convert_type(
        yd.reshape((E + 1) * cap, DW, 2), jnp.int32)
    out32 = _combine(yd32, gidx.reshape(N * K_TOP), N=N, D=DW)
    out = lax.bitcast_convert_type(out32, jnp.bfloat16).reshape(N, D)
    return out.astype(jnp.float32).reshape(B, T, D), aux.reshape(())


# speedup vs baseline: 2.4270x; 2.4270x over previous
import functools
import math

import jax
import jax.numpy as jnp
from jax import lax
from jax.experimental import pallas as pl
from jax.experimental.pallas import tpu as pltpu
from jax.experimental.pallas import tpu_sc as plsc

K_TOP = 2
CAP_FACTOR = 1.0
AUX_COEF = 0.01


def _routing_body(x_ref, wg_ref, scidx_ref, gidx_ref, weff_ref, aux_ref,
                  carry_ref, pacc_ref, *, cap, E, TM, N):
    i = pl.program_id(0)

    @pl.when(i == 0)
    def _init():
        carry_ref[...] = jnp.zeros_like(carry_ref)
        pacc_ref[...] = jnp.zeros_like(pacc_ref)

    x = x_ref[...]
    wg = wg_ref[...]
    logits = jnp.dot(x, wg, preferred_element_type=jnp.float32)
    mx = jnp.max(logits, axis=1, keepdims=True)
    ex = jnp.exp(logits - mx)
    probs = ex / jnp.sum(ex, axis=1, keepdims=True)

    iota_e = lax.broadcasted_iota(jnp.int32, (TM, E), 1)
    m1 = jnp.max(probs, axis=1, keepdims=True)
    am1 = jnp.min(jnp.where(probs >= m1, iota_e, E), axis=1, keepdims=True)
    masked = jnp.where(iota_e == am1, -jnp.inf, probs)
    m2 = jnp.max(masked, axis=1, keepdims=True)
    am2 = jnp.min(jnp.where(masked >= m2, iota_e, E), axis=1, keepdims=True)
    s = m1 + m2
    w0 = m1 / (s + 1e-9)
    w1 = m2 / (s + 1e-9)

    oh0 = (iota_e == am1).astype(jnp.float32)
    oh1 = (iota_e == am2).astype(jnp.float32)
    c = oh0 + oh1

    r_i = lax.broadcasted_iota(jnp.int32, (TM, TM), 0)
    c_i = lax.broadcasted_iota(jnp.int32, (TM, TM), 1)
    ltri = (c_i < r_i).astype(jnp.float32)
    excl = jnp.dot(ltri, c, preferred_element_type=jnp.float32)
    carry = carry_ref[...]
    base = excl + carry
    pos0 = jnp.sum(base * oh0, axis=1, keepdims=True).astype(jnp.int32)
    pos1 = jnp.sum(base * oh1, axis=1, keepdims=True).astype(jnp.int32)
    kept0 = pos0 < cap
    kept1 = pos1 < cap

    tok = i * TM + lax.broadcasted_iota(jnp.int32, (TM, 1), 0)
    dump0 = E * cap + (2 * tok) % 16
    dump1 = E * cap + (2 * tok + 1) % 16
    sc0 = jnp.where(kept0, am1 * cap + pos0, dump0)
    sc1 = jnp.where(kept1, am2 * cap + pos1, dump1)
    zr = E * cap
    g0 = jnp.where(kept0, am1 * cap + pos0, zr)
    g1 = jnp.where(kept1, am2 * cap + pos1, zr)
    we0 = w0 * kept0.astype(jnp.float32)
    we1 = w1 * kept1.astype(jnp.float32)

    scidx_ref[...] = jnp.concatenate([sc0, sc1], axis=1)
    gidx_ref[...] = jnp.concatenate([g0, g1], axis=1)
    weff_ref[...] = jnp.concatenate([we0, we1], axis=1)

    new_carry = carry + jnp.sum(c, axis=0, keepdims=True)
    carry_ref[...] = new_carry
    pacc = pacc_ref[...] + jnp.sum(probs, axis=0, keepdims=True)
    pacc_ref[...] = pacc

    load = jnp.minimum(new_carry, float(cap))
    fa = load / (jnp.sum(load) + 1e-9)
    fe = pacc / N
    fe = fe / (jnp.sum(fe) + 1e-9)
    lb = jnp.mean((fe - fa) ** 2)
    aux_ref[...] = jnp.full((1, 1), AUX_COEF, jnp.float32) * lb


def _routing(xf, w_gate, *, cap, interpret=False):
    N, D = xf.shape
    E = w_gate.shape[1]
    TM = 256
    body = functools.partial(_routing_body, cap=cap, E=E, TM=TM, N=N)
    return pl.pallas_call(
        body,
        grid=(N // TM,),
        in_specs=[
            pl.BlockSpec((TM, D), lambda i: (i, 0)),
            pl.BlockSpec((D, E), lambda i: (0, 0)),
        ],
        out_specs=[
            pl.BlockSpec((TM, K_TOP), lambda i: (i, 0)),
            pl.BlockSpec((TM, K_TOP), lambda i: (i, 0)),
            pl.BlockSpec((TM, K_TOP), lambda i: (i, 0)),
            pl.BlockSpec((1, 1), lambda i: (0, 0)),
        ],
        out_shape=[
            jax.ShapeDtypeStruct((N, K_TOP), jnp.int32),
            jax.ShapeDtypeStruct((N, K_TOP), jnp.int32),
            jax.ShapeDtypeStruct((N, K_TOP), jnp.float32),
            jax.ShapeDtypeStruct((1, 1), jnp.float32),
        ],
        scratch_shapes=[
            pltpu.VMEM((1, E), jnp.float32),
            pltpu.VMEM((1, E), jnp.float32),
        ],
        compiler_params=pltpu.CompilerParams(
            dimension_semantics=("arbitrary",)),
        interpret=interpret,
    )(xf, w_gate)


def _dispatch(scidx, weff, xbf, *, n_rows):
    N, D = xbf.shape
    NK = scidx.shape[0]
    info = plsc.get_sparse_core_info()
    NW = info.num_cores * info.num_subcores
    NC = info.num_cores
    TBL = n_rows + 16
    rows_per_tile = n_rows // NW
    CH = 32
    NCH = rows_per_tile // CH

    mesh = plsc.VectorSubcoreMesh(core_axis_name="c", subcore_axis_name="s")

    @functools.partial(
        pl.kernel,
        mesh=mesh,
        out_type=[
            jax.ShapeDtypeStruct((n_rows, D), jnp.float32),
            jax.ShapeDtypeStruct((n_rows,), jnp.float32),
        ],
        scratch_types=[
            pltpu.VMEM((NK,), jnp.int32),
            pltpu.VMEM((NK,), jnp.float32),
            pltpu.VMEM((TBL,), jnp.int32),
            pltpu.VMEM((TBL,), jnp.float32),
            pltpu.VMEM((2, CH, D), jnp.float32),
            pltpu.SemaphoreType.DMA,
            pltpu.SemaphoreType.DMA,
            pltpu.SemaphoreType.DMA,
            pltpu.SemaphoreType.DMA,
        ],
        compiler_params=pltpu.CompilerParams(needs_layout_passes=False),
    )
    def k(scidx_hbm, weff_hbm, x_hbm, xd_hbm, wtab_hbm,
          idx_v, wf_v, tbl_v, wtb_v, rows_v, sg0, sg1, sw0, sw1):
        wid = lax.axis_index("s") * NC + lax.axis_index("c")
        pltpu.sync_copy(scidx_hbm, idx_v)
        pltpu.sync_copy(weff_hbm, wf_v)

        zi16 = jnp.zeros((16,), jnp.int32)
        zf16 = jnp.zeros((16,), jnp.float32)

        def zbody(j, _):
            tbl_v[pl.ds(j * 16, 16)] = zi16
            wtb_v[pl.ds(j * 16, 16)] = zf16
            return 0

        lax.fori_loop(0, TBL // 16, zbody, 0, unroll=4)

        def sbody(j, _):
            idx = idx_v[pl.ds(j * 16, 16)]
            vals = (lax.iota(jnp.int32, 16) + j * 16) // K_TOP
            plsc.store_scatter(tbl_v, [idx], vals)
            plsc.store_scatter(wtb_v, [idx], wf_v[pl.ds(j * 16, 16)])
            return 0

        lax.fori_loop(0, NK // 16, sbody, 0, unroll=4)

        base = wid * rows_per_tile
        pltpu.sync_copy(wtb_v.at[pl.ds(base, rows_per_tile)],
                        wtab_hbm.at[pl.ds(base, rows_per_tile)])

        sg = (sg0, sg1)
        sw = (sw0, sw1)

        def gstart(ci, b):
            return pltpu.async_copy(
                x_hbm.at[tbl_v.at[pl.ds(base + ci * CH, CH)]],
                rows_v.at[b], sg[b])

        gc = [None, None]
        wc = [None, None]
        gc[0] = gstart(0, 0)
        for ci in range(NCH):
            b = ci % 2
            nb = (ci + 1) % 2
            gc[b].wait()
            if ci + 1 < NCH:
                if wc[nb] is not None:
                    wc[nb].wait()
                gc[nb] = gstart(ci + 1, nb)
            wc[b] = pltpu.async_copy(
                rows_v.at[b], xd_hbm.at[pl.ds(base + ci * CH, CH)], sw[b])
        for b in range(2):
            if wc[b] is not None:
                wc[b].wait()

    return k(scidx, weff, xbf)


def _ffn_body(x_ref, w_ref, w1_ref, b1_ref, w2_ref, b2_ref, out_ref,
              acc_ref, *, E, NF):
    e = pl.program_id(0)
    f = pl.program_id(1)

    @pl.when(e == E)
    def _zeros():
        out_ref[0] = jnp.zeros_like(out_ref[0])

    @pl.when(e < E)
    def _compute():
        x = x_ref[0]
        h = jnp.dot(x, w1_ref[0], preferred_element_type=jnp.float32)
        h = h + b1_ref[0, 0]
        h = 0.5 * h * (1.0 + lax.erf(h * (1.0 / math.sqrt(2.0))))
        y = jnp.dot(h, w2_ref[0], preferred_element_type=jnp.float32)

        @pl.when(f == 0)
        def _first():
            acc_ref[...] = y + b2_ref[0]

        @pl.when(f > 0)
        def _rest():
            acc_ref[...] = acc_ref[...] + y

        @pl.when(f == NF - 1)
        def _emit():
            yw = acc_ref[...] * w_ref[0]
            dw = yw.shape[1] // 2
            lo = lax.bitcast_convert_type(
                yw[:, :dw].astype(jnp.bfloat16), jnp.uint16)
            hi = lax.bitcast_convert_type(
                yw[:, dw:].astype(jnp.bfloat16), jnp.uint16)
            word = lo.astype(jnp.uint32) | (hi.astype(jnp.uint32) << 16)
            out_ref[0] = lax.bitcast_convert_type(word, jnp.int32)


def _ffn(xd, wtab, W1, b1, W2, b2, *, interpret=False):
    E, cap, D = xd.shape
    FF = W1.shape[2]
    TF = 1024 if FF % 1024 == 0 else FF
    NF = FF // TF
    ce = lambda e: jnp.minimum(e, E - 1)
    body = functools.partial(_ffn_body, E=E, NF=NF)
    return pl.pallas_call(
        body,
        grid=(E + 1, NF),
        in_specs=[
            pl.BlockSpec((1, cap, D), lambda e, f: (ce(e), 0, 0)),
            pl.BlockSpec((1, cap, 1), lambda e, f: (ce(e), 0, 0)),
            pl.BlockSpec((1, D, TF), lambda e, f: (ce(e), 0, f)),
            pl.BlockSpec((1, 1, 1, TF), lambda e, f: (ce(e), f, 0, 0)),
            pl.BlockSpec((1, TF, D), lambda e, f: (ce(e), f, 0)),
            pl.BlockSpec((1, 1, D), lambda e, f: (ce(e), 0, 0)),
        ],
        out_specs=pl.BlockSpec((1, cap, D // 2), lambda e, f: (e, 0, 0)),
        out_shape=jax.ShapeDtypeStruct((E + 1, cap, D // 2), jnp.int32),
        scratch_shapes=[pltpu.VMEM((cap, D), jnp.float32)],
        compiler_params=pltpu.CompilerParams(
            dimension_semantics=("arbitrary", "arbitrary")),
        interpret=interpret,
    )(xd, wtab.reshape(E, cap, 1), W1, b1.reshape(E, NF, 1, TF), W2,
      b2.reshape(E, 1, D))


def _combine(yd, gidx, *, N, D):
    info = plsc.get_sparse_core_info()
    NW = info.num_cores * info.num_subcores
    NC = info.num_cores
    TPT = N // NW
    CT = 16
    NCH = TPT // CT
    NV = D // 16

    mesh = plsc.VectorSubcoreMesh(core_axis_name="c", subcore_axis_name="s")

    @functools.partial(
        pl.kernel,
        mesh=mesh,
        out_type=jax.ShapeDtypeStruct((N, D), jnp.int32),
        scratch_types=[
            pltpu.VMEM((K_TOP * TPT,), jnp.int32),
            pltpu.VMEM((2, K_TOP * CT, D), jnp.int32),
            pltpu.VMEM((2, CT, D), jnp.int32),
            pltpu.SemaphoreType.DMA,
            pltpu.SemaphoreType.DMA,
            pltpu.SemaphoreType.DMA,
            pltpu.SemaphoreType.DMA,
        ],
        compiler_params=pltpu.CompilerParams(needs_layout_passes=False),
    )
    def k(yd_hbm, gidx_hbm, out_hbm, idx_v, rows_v, out_v, sg0, sg1, sw0, sw1):
        wid = lax.axis_index("s") * NC + lax.axis_index("c")
        tbase = wid * TPT
        pltpu.sync_copy(gidx_hbm.at[pl.ds(K_TOP * tbase, K_TOP * TPT)], idx_v)

        sg = (sg0, sg1)
        sw = (sw0, sw1)

        def gstart(ci, b):
            return pltpu.async_copy(
                yd_hbm.at[idx_v.at[pl.ds(ci * K_TOP * CT, K_TOP * CT)]],
                rows_v.at[b], sg[b])

        gc = [None, None]
        wc = [None, None]
        gc[0] = gstart(0, 0)
        for ci in range(NCH):
            b = ci % 2
            nb = (ci + 1) % 2
            gc[b].wait()
            if ci + 1 < NCH:
                gc[nb] = gstart(ci + 1, nb)
            if wc[b] is not None:
                wc[b].wait()

            def tbody(t, _, b=b):
                def vbody(v, _):
                    r0 = plsc.bitcast(
                        rows_v[b, 2 * t, pl.ds(v * 16, 16)], jnp.bfloat16)
                    r1 = plsc.bitcast(
                        rows_v[b, 2 * t + 1, pl.ds(v * 16, 16)], jnp.bfloat16)
                    out_v[b, t, pl.ds(v * 16, 16)] = plsc.bitcast(
                        r0 + r1, jnp.int32)
                    return 0

                lax.fori_loop(0, NV, vbody, 0, unroll=8)
                return 0

            lax.fori_loop(0, CT, tbody, 0)
            wc[b] = pltpu.async_copy(
                out_v.at[b], out_hbm.at[pl.ds(tbase + ci * CT, CT)], sw[b])
        for b in range(2):
            if wc[b] is not None:
                wc[b].wait()

    return k(yd, gidx)


def kernel(x, w_gate, W1, b1, W2, b2):
    B, T, D = x.shape
    N = B * T
    E = w_gate.shape[1]
    cap = max(1, int(CAP_FACTOR * N * max(1, K_TOP) / E + 0.9999))
    xf = x.reshape(N, D)

    scidx, gidx, weff, aux = _routing(xf, w_gate, cap=cap)
    DW = D // 2
    xd, wtab = _dispatch(scidx.reshape(N * K_TOP), weff.reshape(N * K_TOP),
                         xf, n_rows=E * cap)
    yd32 = _ffn(xd.reshape(E, cap, D), wtab.reshape(E, cap), W1, b1, W2, b2)
    out32 = _combine(yd32.reshape((E + 1) * cap, DW),
                     gidx.reshape(N * K_TOP), N=N, D=DW)
    ou = lax.bitcast_convert_type(out32, jnp.uint32)
    lo = lax.bitcast_convert_type((ou & 0xFFFF).astype(jnp.uint16),
                                  jnp.bfloat16).astype(jnp.float32)
    hi = lax.bitcast_convert_type((ou >> 16).astype(jnp.uint16),
                                  jnp.bfloat16).astype(jnp.float32)
    out = jnp.concatenate([lo, hi], axis=1)
    return out.reshape(B, T, D), aux.reshape(())


# speedup vs baseline: 2.5158x; 1.0366x over previous
import functools
import math

import jax
import jax.numpy as jnp
from jax import lax
from jax.experimental import pallas as pl
from jax.experimental.pallas import tpu as pltpu
from jax.experimental.pallas import tpu_sc as plsc

K_TOP = 2
CAP_FACTOR = 1.0
AUX_COEF = 0.01


def _routing_body(x_ref, wg_ref, scidx_ref, gidx_ref, weff_ref, aux_ref,
                  carry_ref, pacc_ref, *, cap, E, TM, N):
    i = pl.program_id(0)

    @pl.when(i == 0)
    def _init():
        carry_ref[...] = jnp.zeros_like(carry_ref)
        pacc_ref[...] = jnp.zeros_like(pacc_ref)

    x = x_ref[...]
    wg = wg_ref[...]
    logits = jnp.dot(x, wg, preferred_element_type=jnp.float32)
    mx = jnp.max(logits, axis=1, keepdims=True)
    ex = jnp.exp(logits - mx)
    probs = ex / jnp.sum(ex, axis=1, keepdims=True)

    iota_e = lax.broadcasted_iota(jnp.int32, (TM, E), 1)
    m1 = jnp.max(probs, axis=1, keepdims=True)
    am1 = jnp.min(jnp.where(probs >= m1, iota_e, E), axis=1, keepdims=True)
    masked = jnp.where(iota_e == am1, -jnp.inf, probs)
    m2 = jnp.max(masked, axis=1, keepdims=True)
    am2 = jnp.min(jnp.where(masked >= m2, iota_e, E), axis=1, keepdims=True)
    s = m1 + m2
    w0 = m1 / (s + 1e-9)
    w1 = m2 / (s + 1e-9)

    oh0 = (iota_e == am1).astype(jnp.float32)
    oh1 = (iota_e == am2).astype(jnp.float32)
    c = oh0 + oh1

    r_i = lax.broadcasted_iota(jnp.int32, (TM, TM), 0)
    c_i = lax.broadcasted_iota(jnp.int32, (TM, TM), 1)
    ltri = (c_i < r_i).astype(jnp.float32)
    excl = jnp.dot(ltri, c, preferred_element_type=jnp.float32)
    carry = carry_ref[...]
    base = excl + carry
    pos0 = jnp.sum(base * oh0, axis=1, keepdims=True).astype(jnp.int32)
    pos1 = jnp.sum(base * oh1, axis=1, keepdims=True).astype(jnp.int32)
    kept0 = pos0 < cap
    kept1 = pos1 < cap

    tok = i * TM + lax.broadcasted_iota(jnp.int32, (TM, 1), 0)
    dump0 = E * cap + (2 * tok) % 16
    dump1 = E * cap + (2 * tok + 1) % 16
    sc0 = jnp.where(kept0, am1 * cap + pos0, dump0)
    sc1 = jnp.where(kept1, am2 * cap + pos1, dump1)
    zr = E * cap
    g0 = jnp.where(kept0, am1 * cap + pos0, zr)
    g1 = jnp.where(kept1, am2 * cap + pos1, zr)
    we0 = w0 * kept0.astype(jnp.float32)
    we1 = w1 * kept1.astype(jnp.float32)

    scidx_ref[...] = jnp.concatenate([sc0, sc1], axis=1)
    gidx_ref[...] = jnp.concatenate([g0, g1], axis=1)
    weff_ref[...] = jnp.concatenate([we0, we1], axis=1)

    new_carry = carry + jnp.sum(c, axis=0, keepdims=True)
    carry_ref[...] = new_carry
    pacc = pacc_ref[...] + jnp.sum(probs, axis=0, keepdims=True)
    pacc_ref[...] = pacc

    load = jnp.minimum(new_carry, float(cap))
    fa = load / (jnp.sum(load) + 1e-9)
    fe = pacc / N
    fe = fe / (jnp.sum(fe) + 1e-9)
    lb = jnp.mean((fe - fa) ** 2)
    aux_ref[...] = jnp.full((1, 1), AUX_COEF, jnp.float32) * lb


def _routing(xf, w_gate, *, cap, interpret=False):
    N, D = xf.shape
    E = w_gate.shape[1]
    TM = 256
    body = functools.partial(_routing_body, cap=cap, E=E, TM=TM, N=N)
    return pl.pallas_call(
        body,
        grid=(N // TM,),
        in_specs=[
            pl.BlockSpec((TM, D), lambda i: (i, 0)),
            pl.BlockSpec((D, E), lambda i: (0, 0)),
        ],
        out_specs=[
            pl.BlockSpec((TM, K_TOP), lambda i: (i, 0)),
            pl.BlockSpec((TM, K_TOP), lambda i: (i, 0)),
            pl.BlockSpec((TM, K_TOP), lambda i: (i, 0)),
            pl.BlockSpec((1, 1), lambda i: (0, 0)),
        ],
        out_shape=[
            jax.ShapeDtypeStruct((N, K_TOP), jnp.int32),
            jax.ShapeDtypeStruct((N, K_TOP), jnp.int32),
            jax.ShapeDtypeStruct((N, K_TOP), jnp.float32),
            jax.ShapeDtypeStruct((1, 1), jnp.float32),
        ],
        scratch_shapes=[
            pltpu.VMEM((1, E), jnp.float32),
            pltpu.VMEM((1, E), jnp.float32),
        ],
        compiler_params=pltpu.CompilerParams(
            dimension_semantics=("arbitrary",)),
        interpret=interpret,
    )(xf, w_gate)


def _dispatch(scidx, weff, xbf, *, n_rows):
    N, D = xbf.shape
    NK = scidx.shape[0]
    info = plsc.get_sparse_core_info()
    NW = info.num_cores * info.num_subcores
    NC = info.num_cores
    TBL = n_rows + 16
    rows_per_tile = n_rows // NW
    CH = 32
    NCH = rows_per_tile // CH

    mesh = plsc.VectorSubcoreMesh(core_axis_name="c", subcore_axis_name="s")

    @functools.partial(
        pl.kernel,
        mesh=mesh,
        out_type=[
            jax.ShapeDtypeStruct((n_rows, D), jnp.float32),
            jax.ShapeDtypeStruct((n_rows,), jnp.float32),
        ],
        scratch_types=[
            pltpu.VMEM((NK,), jnp.int32),
            pltpu.VMEM((NK,), jnp.float32),
            pltpu.VMEM((TBL,), jnp.int32),
            pltpu.VMEM((TBL,), jnp.float32),
            pltpu.VMEM((2, CH, D), jnp.float32),
            pltpu.SemaphoreType.DMA,
            pltpu.SemaphoreType.DMA,
            pltpu.SemaphoreType.DMA,
            pltpu.SemaphoreType.DMA,
        ],
        compiler_params=pltpu.CompilerParams(needs_layout_passes=False),
    )
    def k(scidx_hbm, weff_hbm, x_hbm, xd_hbm, wtab_hbm,
          idx_v, wf_v, tbl_v, wtb_v, rows_v, sg0, sg1, sw0, sw1):
        wid = lax.axis_index("s") * NC + lax.axis_index("c")
        pltpu.sync_copy(scidx_hbm, idx_v)
        pltpu.sync_copy(weff_hbm, wf_v)

        zi16 = jnp.zeros((16,), jnp.int32)
        zf16 = jnp.zeros((16,), jnp.float32)

        def zbody(j, _):
            tbl_v[pl.ds(j * 16, 16)] = zi16
            wtb_v[pl.ds(j * 16, 16)] = zf16
            return 0

        lax.fori_loop(0, TBL // 16, zbody, 0, unroll=4)

        def sbody(j, _):
            idx = idx_v[pl.ds(j * 16, 16)]
            vals = (lax.iota(jnp.int32, 16) + j * 16) // K_TOP
            plsc.store_scatter(tbl_v, [idx], vals)
            plsc.store_scatter(wtb_v, [idx], wf_v[pl.ds(j * 16, 16)])
            return 0

        lax.fori_loop(0, NK // 16, sbody, 0, unroll=4)

        base = wid * rows_per_tile
        pltpu.sync_copy(wtb_v.at[pl.ds(base, rows_per_tile)],
                        wtab_hbm.at[pl.ds(base, rows_per_tile)])

        sg = (sg0, sg1)
        sw = (sw0, sw1)

        def gstart(ci, b):
            return pltpu.async_copy(
                x_hbm.at[tbl_v.at[pl.ds(base + ci * CH, CH)]],
                rows_v.at[b], sg[b])

        gc = [None, None]
        wc = [None, None]
        gc[0] = gstart(0, 0)
        for ci in range(NCH):
            b = ci % 2
            nb = (ci + 1) % 2
            gc[b].wait()
            if ci + 1 < NCH:
                if wc[nb] is not None:
                    wc[nb].wait()
                gc[nb] = gstart(ci + 1, nb)
            wc[b] = pltpu.async_copy(
                rows_v.at[b], xd_hbm.at[pl.ds(base + ci * CH, CH)], sw[b])
        for b in range(2):
            if wc[b] is not None:
                wc[b].wait()

    return k(scidx, weff, xbf)


def _ffn_body(x_ref, w_ref, w1_ref, b1_ref, w2_ref, b2_ref, out_ref,
              acc_ref, *, E, NF):
    e = pl.program_id(0)
    f = pl.program_id(1)

    @pl.when(e == E)
    def _zeros():
        out_ref[0] = jnp.zeros_like(out_ref[0])

    @pl.when(e < E)
    def _compute():
        x = x_ref[0]
        h = jnp.dot(x, w1_ref[0], preferred_element_type=jnp.float32)
        h = h + b1_ref[0, 0]
        h = 0.5 * h * (1.0 + lax.erf(h * (1.0 / math.sqrt(2.0))))
        y = jnp.dot(h, w2_ref[0], preferred_element_type=jnp.float32)

        @pl.when(f == 0)
        def _first():
            acc_ref[...] = y + b2_ref[0]

        @pl.when(f > 0)
        def _rest():
            acc_ref[...] = acc_ref[...] + y

        @pl.when(f == NF - 1)
        def _emit():
            yw = acc_ref[...] * w_ref[0]
            dw = yw.shape[1] // 2
            lo = lax.bitcast_convert_type(
                yw[:, :dw].astype(jnp.bfloat16), jnp.uint16)
            hi = lax.bitcast_convert_type(
                yw[:, dw:].astype(jnp.bfloat16), jnp.uint16)
            word = lo.astype(jnp.uint32) | (hi.astype(jnp.uint32) << 16)
            out_ref[0] = lax.bitcast_convert_type(word, jnp.int32)


def _ffn(xd, wtab, W1, b1, W2, b2, *, interpret=False):
    E, cap, D = xd.shape
    FF = W1.shape[2]
    TF = 2048 if FF % 2048 == 0 else FF
    NF = FF // TF
    ce = lambda e: jnp.minimum(e, E - 1)
    body = functools.partial(_ffn_body, E=E, NF=NF)
    return pl.pallas_call(
        body,
        grid=(E + 1, NF),
        in_specs=[
            pl.BlockSpec((1, cap, D), lambda e, f: (ce(e), 0, 0)),
            pl.BlockSpec((1, cap, 1), lambda e, f: (ce(e), 0, 0)),
            pl.BlockSpec((1, D, TF), lambda e, f: (ce(e), 0, f)),
            pl.BlockSpec((1, 1, 1, TF), lambda e, f: (ce(e), f, 0, 0)),
            pl.BlockSpec((1, TF, D), lambda e, f: (ce(e), f, 0)),
            pl.BlockSpec((1, 1, D), lambda e, f: (ce(e), 0, 0)),
        ],
        out_specs=pl.BlockSpec((1, cap, D // 2), lambda e, f: (e, 0, 0)),
        out_shape=jax.ShapeDtypeStruct((E + 1, cap, D // 2), jnp.int32),
        scratch_shapes=[pltpu.VMEM((cap, D), jnp.float32)],
        compiler_params=pltpu.CompilerParams(
            dimension_semantics=("arbitrary", "arbitrary")),
        interpret=interpret,
    )(xd, wtab.reshape(E, cap, 1), W1, b1.reshape(E, NF, 1, TF), W2,
      b2.reshape(E, 1, D))


def _combine(yd, gidx, *, N, D):
    info = plsc.get_sparse_core_info()
    NW = info.num_cores * info.num_subcores
    NC = info.num_cores
    TPT = N // NW
    CT = 16
    NCH = TPT // CT
    NV = D // 16

    mesh = plsc.VectorSubcoreMesh(core_axis_name="c", subcore_axis_name="s")

    @functools.partial(
        pl.kernel,
        mesh=mesh,
        out_type=jax.ShapeDtypeStruct((N, D), jnp.int32),
        scratch_types=[
            pltpu.VMEM((K_TOP * TPT,), jnp.int32),
            pltpu.VMEM((2, K_TOP * CT, D), jnp.int32),
            pltpu.VMEM((2, CT, D), jnp.int32),
            pltpu.SemaphoreType.DMA,
            pltpu.SemaphoreType.DMA,
            pltpu.SemaphoreType.DMA,
            pltpu.SemaphoreType.DMA,
        ],
        compiler_params=pltpu.CompilerParams(needs_layout_passes=False),
    )
    def k(yd_hbm, gidx_hbm, out_hbm, idx_v, rows_v, out_v, sg0, sg1, sw0, sw1):
        wid = lax.axis_index("s") * NC + lax.axis_index("c")
        tbase = wid * TPT
        pltpu.sync_copy(gidx_hbm.at[pl.ds(K_TOP * tbase, K_TOP * TPT)], idx_v)

        sg = (sg0, sg1)
        sw = (sw0, sw1)

        def gstart(ci, b):
            return pltpu.async_copy(
                yd_hbm.at[idx_v.at[pl.ds(ci * K_TOP * CT, K_TOP * CT)]],
                rows_v.at[b], sg[b])

        gc = [None, None]
        wc = [None, None]
        gc[0] = gstart(0, 0)
        for ci in range(NCH):
            b = ci % 2
            nb = (ci + 1) % 2
            gc[b].wait()
            if ci + 1 < NCH:
                gc[nb] = gstart(ci + 1, nb)
            if wc[b] is not None:
                wc[b].wait()

            def tbody(t, _, b=b):
                def vbody(v, _):
                    r0 = plsc.bitcast(
                        rows_v[b, 2 * t, pl.ds(v * 16, 16)], jnp.bfloat16)
                    r1 = plsc.bitcast(
                        rows_v[b, 2 * t + 1, pl.ds(v * 16, 16)], jnp.bfloat16)
                    out_v[b, t, pl.ds(v * 16, 16)] = plsc.bitcast(
                        r0 + r1, jnp.int32)
                    return 0

                lax.fori_loop(0, NV, vbody, 0, unroll=8)
                return 0

            lax.fori_loop(0, CT, tbody, 0)
            wc[b] = pltpu.async_copy(
                out_v.at[b], out_hbm.at[pl.ds(tbase + ci * CT, CT)], sw[b])
        for b in range(2):
            if wc[b] is not None:
                wc[b].wait()

    return k(yd, gidx)


def kernel(x, w_gate, W1, b1, W2, b2):
    B, T, D = x.shape
    N = B * T
    E = w_gate.shape[1]
    cap = max(1, int(CAP_FACTOR * N * max(1, K_TOP) / E + 0.9999))
    xf = x.reshape(N, D)

    scidx, gidx, weff, aux = _routing(xf, w_gate, cap=cap)
    DW = D // 2
    xd, wtab = _dispatch(scidx.reshape(N * K_TOP), weff.reshape(N * K_TOP),
                         xf, n_rows=E * cap)
    yd32 = _ffn(xd.reshape(E, cap, D), wtab.reshape(E, cap), W1, b1, W2, b2)
    out32 = _combine(yd32.reshape((E + 1) * cap, DW),
                     gidx.reshape(N * K_TOP), N=N, D=DW)
    ou = lax.bitcast_convert_type(out32, jnp.uint32)
    lo = lax.bitcast_convert_type((ou & 0xFFFF).astype(jnp.uint16),
                                  jnp.bfloat16).astype(jnp.float32)
    hi = lax.bitcast_convert_type((ou >> 16).astype(jnp.uint16),
                                  jnp.bfloat16).astype(jnp.float32)
    out = jnp.concatenate([lo, hi], axis=1)
    return out.reshape(B, T, D), aux.reshape(())
